# baseline (speedup 1.0000x reference)
import functools

import jax
import jax.numpy as jnp
from jax import lax
from jax.experimental import pallas as pl
from jax.experimental.pallas import tpu as pltpu
from jax.experimental.pallas import tpu_sc as plsc

BATCH = 4096
EMB_DIM = 64
IMG_FLAT = 3 * 64 * 64
OUT_DIM = IMG_FLAT + EMB_DIM

_NC = 2
_NS = 16
_NW = _NC * _NS
_B_PER_W = BATCH // _NW


def _sc_gather(idx, table):
    mesh = plsc.VectorSubcoreMesh(core_axis_name="c", subcore_axis_name="s")

    @functools.partial(
        pl.kernel,
        mesh=mesh,
        out_type=jax.ShapeDtypeStruct((BATCH, EMB_DIM), jnp.float32),
        scratch_types=[
            pltpu.VMEM((_B_PER_W,), jnp.int32),
            pltpu.VMEM((_B_PER_W, EMB_DIM), jnp.float32),
            pltpu.SemaphoreType.DMA,
        ],
        compiler_params=pltpu.CompilerParams(use_tc_tiling_on_sc=False),
    )
    def gather_kernel(idx_hbm, table_hbm, out_hbm, idx_v, rows_v, sem):
        wid = lax.axis_index("s") * _NC + lax.axis_index("c")
        base = wid * _B_PER_W
        pltpu.sync_copy(idx_hbm.at[pl.ds(base, _B_PER_W)], idx_v)
        pltpu.async_copy(table_hbm.at[idx_v], rows_v, sem).wait()
        pltpu.sync_copy(rows_v, out_hbm.at[pl.ds(base, _B_PER_W)])

    return gather_kernel(idx, table)


_CHF = 768
_NCHF = IMG_FLAT // _CHF
_NBUF = 4
_LOOK = 2


def _assemble_body(imgT_hbm, membT_hbm, outT_hbm, buf, membbuf,
                   in_sems, out_sems, memb_sem):
    def in_cp(g):
        slot = lax.rem(g, _NBUF)
        return pltpu.make_async_copy(
            imgT_hbm.at[pl.ds(g * _CHF, _CHF), :],
            buf.at[slot],
            in_sems.at[slot])

    def out_cp(g):
        slot = lax.rem(g, _NBUF)
        return pltpu.make_async_copy(
            buf.at[slot],
            outT_hbm.at[pl.ds(g * _CHF, _CHF), :],
            out_sems.at[slot])

    memb_in = pltpu.make_async_copy(membT_hbm, membbuf, memb_sem)
    memb_in.start()
    for g in range(_LOOK):
        in_cp(g).start()
    memb_in.wait()
    memb_out = pltpu.make_async_copy(
        membbuf, outT_hbm.at[pl.ds(IMG_FLAT, EMB_DIM), :], memb_sem)
    memb_out.start()

    def step(g, carry):
        @pl.when(g + _LOOK < _NCHF)
        def _prefetch():
            in_cp(g + _LOOK).start()

        in_cp(g).wait()

        @pl.when(g >= _NBUF - _LOOK)
        def _wait_prev_write():
            out_cp(g - (_NBUF - _LOOK)).wait()

        out_cp(g).start()
        return carry

    lax.fori_loop(0, _NCHF, step, 0)

    def drain(g, carry):
        out_cp(g).wait()
        return carry

    lax.fori_loop(_NCHF - (_NBUF - _LOOK), _NCHF, drain, 0)
    memb_out.wait()


def _tc_assemble(imgT, membT):
    return pl.pallas_call(
        _assemble_body,
        in_specs=[
            pl.BlockSpec(memory_space=pl.ANY),
            pl.BlockSpec(memory_space=pl.ANY),
        ],
        out_specs=pl.BlockSpec(memory_space=pl.ANY),
        out_shape=jax.ShapeDtypeStruct((OUT_DIM, BATCH), jnp.float32),
        scratch_shapes=[
            pltpu.VMEM((_NBUF, _CHF, BATCH), jnp.float32),
            pltpu.VMEM((EMB_DIM, BATCH), jnp.float32),
            pltpu.SemaphoreType.DMA((_NBUF,)),
            pltpu.SemaphoreType.DMA((_NBUF,)),
            pltpu.SemaphoreType.DMA,
        ],
        compiler_params=pltpu.CompilerParams(
            vmem_limit_bytes=110 * 1024 * 1024,
        ),
    )(imgT, membT)


_N_DIRECT = 8
_CH_DIRECT = IMG_FLAT // _N_DIRECT


def _img_copy_body(imgT_hbm, outT_hbm, sems):
    copies = [
        pltpu.make_async_copy(
            imgT_hbm.at[pl.ds(c * _CH_DIRECT, _CH_DIRECT), :],
            outT_hbm.at[pl.ds(c * _CH_DIRECT, _CH_DIRECT), :],
            sems.at[c])
        for c in range(_N_DIRECT)
    ]
    for cp in copies:
        cp.start()
    for cp in copies:
        cp.wait()


def _img_copy(imgT):
    return pl.pallas_call(
        _img_copy_body,
        in_specs=[pl.BlockSpec(memory_space=pl.ANY)],
        out_specs=pl.BlockSpec(memory_space=pl.ANY),
        out_shape=jax.ShapeDtypeStruct((OUT_DIM, BATCH), jnp.float32),
        scratch_shapes=[pltpu.SemaphoreType.DMA((_N_DIRECT,))],
    )(imgT)


def _tail_body(outT_in, membT_hbm, outT_hbm, membbuf, sem):
    del outT_in
    cp = pltpu.make_async_copy(membT_hbm, membbuf, sem)
    cp.start()
    cp.wait()
    cp2 = pltpu.make_async_copy(
        membbuf, outT_hbm.at[pl.ds(IMG_FLAT, EMB_DIM), :], sem)
    cp2.start()
    cp2.wait()


def _tail_write(outT, membT):
    return pl.pallas_call(
        _tail_body,
        in_specs=[
            pl.BlockSpec(memory_space=pl.ANY),
            pl.BlockSpec(memory_space=pl.ANY),
        ],
        out_specs=pl.BlockSpec(memory_space=pl.ANY),
        out_shape=jax.ShapeDtypeStruct((OUT_DIM, BATCH), jnp.float32),
        input_output_aliases={0: 0},
        scratch_shapes=[
            pltpu.VMEM((EMB_DIM, BATCH), jnp.float32),
            pltpu.SemaphoreType.DMA,
        ],
    )(outT, membT)


def kernel(image, mission, emb):
    idx = mission.astype(jnp.int32)
    memb = _sc_gather(idx, emb)
    imgT = image.astype(jnp.float32).transpose(1, 2, 3, 0).reshape(IMG_FLAT, BATCH)
    outT = _img_copy(imgT)
    outT = _tail_write(outT, memb.T)
    return outT.T


# speedup vs baseline: 29.5229x; 29.5229x over previous
import functools

import jax
import jax.numpy as jnp
from jax import lax
from jax.experimental import pallas as pl
from jax.experimental.pallas import tpu as pltpu
from jax.experimental.pallas import tpu_sc as plsc

BATCH = 4096
EMB_DIM = 64
IMG_FLAT = 3 * 64 * 64
OUT_DIM = IMG_FLAT + EMB_DIM

_NC = 2
_NS = 16
_NW = _NC * _NS
_B_PER_W = BATCH // _NW


def _sc_gather(idx, table):
    mesh = plsc.VectorSubcoreMesh(core_axis_name="c", subcore_axis_name="s")

    @functools.partial(
        pl.kernel,
        mesh=mesh,
        out_type=jax.ShapeDtypeStruct((BATCH, EMB_DIM), jnp.float32),
        scratch_types=[
            pltpu.VMEM((_B_PER_W,), jnp.int32),
            pltpu.VMEM((_B_PER_W, EMB_DIM), jnp.float32),
            pltpu.SemaphoreType.DMA,
        ],
        compiler_params=pltpu.CompilerParams(use_tc_tiling_on_sc=False),
    )
    def gather_kernel(idx_hbm, table_hbm, out_hbm, idx_v, rows_v, sem):
        wid = lax.axis_index("s") * _NC + lax.axis_index("c")
        base = wid * _B_PER_W
        pltpu.sync_copy(idx_hbm.at[pl.ds(base, _B_PER_W)], idx_v)
        pltpu.async_copy(table_hbm.at[idx_v], rows_v, sem).wait()
        pltpu.sync_copy(rows_v, out_hbm.at[pl.ds(base, _B_PER_W)])

    return gather_kernel(idx, table)


_CHF = 768
_NCHF = IMG_FLAT // _CHF
_NBUF = 4
_LOOK = 2


def _img_copy_body(imgT_hbm, outT_hbm, buf, in_sems, out_sems):
    def in_cp(g):
        slot = lax.rem(g, _NBUF)
        return pltpu.make_async_copy(
            imgT_hbm.at[pl.ds(g * _CHF, _CHF), :],
            buf.at[slot],
            in_sems.at[slot])

    def out_cp(g):
        slot = lax.rem(g, _NBUF)
        return pltpu.make_async_copy(
            buf.at[slot],
            outT_hbm.at[pl.ds(g * _CHF, _CHF), :],
            out_sems.at[slot])

    for g in range(_LOOK):
        in_cp(g).start()

    def step(g, carry):
        @pl.when(g + _LOOK < _NCHF)
        def _prefetch():
            in_cp(g + _LOOK).start()

        in_cp(g).wait()

        @pl.when(g >= _NBUF - _LOOK)
        def _wait_prev_write():
            out_cp(g - (_NBUF - _LOOK)).wait()

        out_cp(g).start()
        return carry

    lax.fori_loop(0, _NCHF, step, 0)

    def drain(g, carry):
        out_cp(g).wait()
        return carry

    lax.fori_loop(_NCHF - (_NBUF - _LOOK), _NCHF, drain, 0)


def _img_copy(imgT):
    return pl.pallas_call(
        _img_copy_body,
        in_specs=[pl.BlockSpec(memory_space=pl.ANY)],
        out_specs=pl.BlockSpec(memory_space=pl.ANY),
        out_shape=jax.ShapeDtypeStruct((OUT_DIM, BATCH), jnp.float32),
        scratch_shapes=[
            pltpu.VMEM((_NBUF, _CHF, BATCH), jnp.float32),
            pltpu.SemaphoreType.DMA((_NBUF,)),
            pltpu.SemaphoreType.DMA((_NBUF,)),
        ],
        compiler_params=pltpu.CompilerParams(
            vmem_limit_bytes=110 * 1024 * 1024,
        ),
    )(imgT)


def _tail_body(outT_in, membT_hbm, outT_hbm, membbuf, sem):
    del outT_in
    cp = pltpu.make_async_copy(membT_hbm, membbuf, sem)
    cp.start()
    cp.wait()
    cp2 = pltpu.make_async_copy(
        membbuf, outT_hbm.at[pl.ds(IMG_FLAT, EMB_DIM), :], sem)
    cp2.start()
    cp2.wait()


def _tail_write(outT, membT):
    return pl.pallas_call(
        _tail_body,
        in_specs=[
            pl.BlockSpec(memory_space=pl.ANY),
            pl.BlockSpec(memory_space=pl.ANY),
        ],
        out_specs=pl.BlockSpec(memory_space=pl.ANY),
        out_shape=jax.ShapeDtypeStruct((OUT_DIM, BATCH), jnp.float32),
        input_output_aliases={0: 0},
        scratch_shapes=[
            pltpu.VMEM((EMB_DIM, BATCH), jnp.float32),
            pltpu.SemaphoreType.DMA,
        ],
    )(outT, membT)


def kernel(image, mission, emb):
    idx = mission.astype(jnp.int32)
    memb = _sc_gather(idx, emb)
    imgT = image.astype(jnp.float32).transpose(1, 2, 3, 0).reshape(IMG_FLAT, BATCH)
    outT = _img_copy(imgT)
    outT = _tail_write(outT, memb.T)
    return outT.T


# speedup vs baseline: 29.5258x; 1.0001x over previous
import functools

import jax
import jax.numpy as jnp
from jax import lax
from jax.experimental import pallas as pl
from jax.experimental.pallas import tpu as pltpu
from jax.experimental.pallas import tpu_sc as plsc

BATCH = 4096
EMB_DIM = 64
IMG_FLAT = 3 * 64 * 64
OUT_DIM = IMG_FLAT + EMB_DIM

_NC = 2
_NS = 16
_NW = _NC * _NS
_B_PER_W = BATCH // _NW


def _sc_gather(idx, table):
    mesh = plsc.VectorSubcoreMesh(core_axis_name="c", subcore_axis_name="s")

    @functools.partial(
        pl.kernel,
        mesh=mesh,
        out_type=jax.ShapeDtypeStruct((BATCH, EMB_DIM), jnp.float32),
        scratch_types=[
            pltpu.VMEM((_B_PER_W,), jnp.int32),
            pltpu.VMEM((_B_PER_W, EMB_DIM), jnp.float32),
            pltpu.SemaphoreType.DMA,
        ],
        compiler_params=pltpu.CompilerParams(use_tc_tiling_on_sc=False),
    )
    def gather_kernel(idx_hbm, table_hbm, out_hbm, idx_v, rows_v, sem):
        wid = lax.axis_index("s") * _NC + lax.axis_index("c")
        base = wid * _B_PER_W
        pltpu.sync_copy(idx_hbm.at[pl.ds(base, _B_PER_W)], idx_v)
        pltpu.async_copy(table_hbm.at[idx_v], rows_v, sem).wait()
        pltpu.sync_copy(rows_v, out_hbm.at[pl.ds(base, _B_PER_W)])

    return gather_kernel(idx, table)


_CHF = 512
_NCHF = IMG_FLAT // _CHF
_NBUF = 7
_LOOK = 3


def _img_copy_body(imgT_hbm, outT_hbm, buf, in_sems, out_sems):
    def in_cp(g):
        slot = lax.rem(g, _NBUF)
        return pltpu.make_async_copy(
            imgT_hbm.at[pl.ds(g * _CHF, _CHF), :],
            buf.at[slot],
            in_sems.at[slot])

    def out_cp(g):
        slot = lax.rem(g, _NBUF)
        return pltpu.make_async_copy(
            buf.at[slot],
            outT_hbm.at[pl.ds(g * _CHF, _CHF), :],
            out_sems.at[slot])

    for g in range(_LOOK):
        in_cp(g).start()

    def step(g, carry):
        @pl.when(g + _LOOK < _NCHF)
        def _prefetch():
            in_cp(g + _LOOK).start()

        in_cp(g).wait()

        @pl.when(g >= _NBUF - _LOOK)
        def _wait_prev_write():
            out_cp(g - (_NBUF - _LOOK)).wait()

        out_cp(g).start()
        return carry

    lax.fori_loop(0, _NCHF, step, 0)

    def drain(g, carry):
        out_cp(g).wait()
        return carry

    lax.fori_loop(_NCHF - (_NBUF - _LOOK), _NCHF, drain, 0)


def _img_copy(imgT):
    return pl.pallas_call(
        _img_copy_body,
        in_specs=[pl.BlockSpec(memory_space=pl.ANY)],
        out_specs=pl.BlockSpec(memory_space=pl.ANY),
        out_shape=jax.ShapeDtypeStruct((OUT_DIM, BATCH), jnp.float32),
        scratch_shapes=[
            pltpu.VMEM((_NBUF, _CHF, BATCH), jnp.float32),
            pltpu.SemaphoreType.DMA((_NBUF,)),
            pltpu.SemaphoreType.DMA((_NBUF,)),
        ],
        compiler_params=pltpu.CompilerParams(
            vmem_limit_bytes=64 * 1024 * 1024,
        ),
    )(imgT)


def _tail_body(outT_in, membT_hbm, outT_hbm, membbuf, sem):
    del outT_in
    cp = pltpu.make_async_copy(membT_hbm, membbuf, sem)
    cp.start()
    cp.wait()
    cp2 = pltpu.make_async_copy(
        membbuf, outT_hbm.at[pl.ds(IMG_FLAT, EMB_DIM), :], sem)
    cp2.start()
    cp2.wait()


def _tail_write(outT, membT):
    return pl.pallas_call(
        _tail_body,
        in_specs=[
            pl.BlockSpec(memory_space=pl.ANY),
            pl.BlockSpec(memory_space=pl.ANY),
        ],
        out_specs=pl.BlockSpec(memory_space=pl.ANY),
        out_shape=jax.ShapeDtypeStruct((OUT_DIM, BATCH), jnp.float32),
        input_output_aliases={0: 0},
        scratch_shapes=[
            pltpu.VMEM((EMB_DIM, BATCH), jnp.float32),
            pltpu.SemaphoreType.DMA,
        ],
    )(outT, membT)


def kernel(image, mission, emb):
    imgT = image.astype(jnp.float32).transpose(1, 2, 3, 0).reshape(IMG_FLAT, BATCH)
    outT = _img_copy(imgT)
    idx = mission.astype(jnp.int32)
    memb = _sc_gather(idx, emb)
    outT = _tail_write(outT, memb.T)
    return outT.T
